# baseline (speedup 1.0000x reference)
import functools

import jax
import jax.numpy as jnp
from jax import lax
from jax.experimental import pallas as pl
from jax.experimental.pallas import tpu as pltpu
from jax.experimental.pallas import tpu_sc as plsc

N = 10000
D = 128
H = 128
C = 64
N_PAD = 10240
E = 320000
CH = 128
N_TILES = 32
E_PAD = 327680
CPT = E_PAD // (N_TILES * CH)
CPG = 2 * CPT
A_C0 = 30
RPT = N_PAD // 16

_SQRT_HALF = 0.7071067811865476


def _gelu(x):
    return x * 0.5 * (1.0 + lax.erf(x * _SQRT_HALF))


def _mesh():
    return plsc.VectorSubcoreMesh(core_axis_name="c", subcore_axis_name="s")


def _unpack_edges(pk_v, row_v, col_v, want_row):
    def body(j, carry):
        for kk in range(CH // 16):
            v = pk_v[j, pl.ds(kk * 16, 16)]
            if want_row:
                row_v[j, pl.ds(kk * 16, 16)] = lax.shift_right_logical(v, 14)
            col_v[j, pl.ds(kk * 16, 16)] = lax.bitwise_and(v, 16383)
        return carry

    lax.fori_loop(0, CPT, body, 0)


def _sc_degree(pk_mat):
    @functools.partial(
        pl.kernel,
        out_type=jax.ShapeDtypeStruct((2, N_PAD, H), jnp.float32),
        mesh=_mesh(),
        scratch_types=[
            pltpu.VMEM((CPT, CH), jnp.int32),
            pltpu.VMEM((CPT, CH), jnp.int32),
            pltpu.VMEM((CH, H), jnp.float32),
            pltpu.VMEM_SHARED((N_PAD, H), jnp.float32),
        ],
    )
    def k(pk_hbm, out_hbm, pk_v, col_v, ones_v, acc):
        c = lax.axis_index("c")
        s = lax.axis_index("s")
        wid = c * 16 + s

        def set_zero(i, carry):
            for kk in range(H // 16):
                ones_v[i, pl.ds(kk * 16, 16)] = jnp.zeros((16,), jnp.float32)
            return carry

        lax.fori_loop(0, CH, set_zero, 0)
        for kk in range(RPT // CH):
            pltpu.sync_copy(ones_v, acc.at[pl.ds(s * RPT + kk * CH, CH)])

        def set_ones(i, carry):
            ones_v[i, pl.ds(0, 16)] = jnp.ones((16,), jnp.float32)
            return carry

        lax.fori_loop(0, CH, set_ones, 0)
        pltpu.sync_copy(pk_hbm.at[pl.ds(wid * CPT, CPT)], pk_v)
        _unpack_edges(pk_v, None, col_v, want_row=False)
        plsc.subcore_barrier()

        def body(j, carry):
            pltpu.sync_copy(ones_v, acc.at[col_v.at[j]], add=True)
            return carry

        lax.fori_loop(0, CPT, body, 0)
        plsc.subcore_barrier()
        pltpu.sync_copy(acc.at[pl.ds(s * RPT, RPT)],
                        out_hbm.at[c, pl.ds(s * RPT, RPT)])

    return k(pk_mat)


def _sc_scatter(g_full, pk_mat):
    @functools.partial(
        pl.kernel,
        out_type=jax.ShapeDtypeStruct((2, N_PAD, H), jnp.float32),
        mesh=_mesh(),
        scratch_types=[
            pltpu.VMEM((CH,), jnp.int32),
            pltpu.VMEM((CH,), jnp.int32),
            pltpu.VMEM((CH,), jnp.int32),
            pltpu.VMEM((CH,), jnp.int32),
            pltpu.VMEM((CH,), jnp.int32),
            pltpu.VMEM((CH,), jnp.int32),
            pltpu.VMEM((CH, H), jnp.float32),
            pltpu.VMEM((CH, H), jnp.float32),
            pltpu.VMEM_SHARED((N_PAD, H), jnp.float32),
            pltpu.SemaphoreType.DMA,
            pltpu.SemaphoreType.DMA,
            pltpu.SemaphoreType.DMA,
            pltpu.SemaphoreType.DMA,
        ],
    )
    def k(g_hbm, pk_hbm, out_hbm,
          pk_c0, pk_c1, row_c0, col_c0, row_c1, col_c1,
          rows_v0, rows_v1, acc, sem0, sem1, psem0, psem1):
        c = lax.axis_index("c")
        s = lax.axis_index("s")
        base = s * CPG + c * A_C0
        cnt = jnp.where(c == 0, A_C0, CPG - A_C0)
        last = base + cnt - 1

        def unpack(pk_c, row_c, col_c):
            for kk in range(CH // 16):
                v = pk_c[pl.ds(kk * 16, 16)]
                row_c[pl.ds(kk * 16, 16)] = lax.shift_right_logical(v, 14)
                col_c[pl.ds(kk * 16, 16)] = lax.bitwise_and(v, 16383)

        def zero_rows(i, carry):
            for kk in range(H // 16):
                rows_v0[i, pl.ds(kk * 16, 16)] = jnp.zeros((16,),
                                                           jnp.float32)
            return carry

        lax.fori_loop(0, CH, zero_rows, 0)
        for kk in range(RPT // CH):
            pltpu.sync_copy(rows_v0, acc.at[pl.ds(s * RPT + kk * CH, CH)])
        plsc.subcore_barrier()

        pltpu.sync_copy(pk_hbm.at[base], pk_c0)
        unpack(pk_c0, row_c0, col_c0)
        pltpu.async_copy(g_hbm.at[row_c0], rows_v0, sem0)
        pltpu.async_copy(pk_hbm.at[base + 1], pk_c1, psem1)

        def body(i, carry):
            j0 = base + i * 2
            pltpu.make_async_copy(pk_hbm.at[j0 + 1], pk_c1, psem1).wait()
            unpack(pk_c1, row_c1, col_c1)
            pltpu.async_copy(g_hbm.at[row_c1], rows_v1, sem1)
            pltpu.async_copy(pk_hbm.at[jnp.minimum(j0 + 2, last)], pk_c0,
                             psem0)
            pltpu.make_async_copy(g_hbm.at[row_c0], rows_v0, sem0).wait()
            pltpu.sync_copy(rows_v0, acc.at[col_c0], add=True)
            pltpu.make_async_copy(pk_hbm.at[j0], pk_c0, psem0).wait()
            unpack(pk_c0, row_c0, col_c0)
            pltpu.async_copy(g_hbm.at[row_c0], rows_v0, sem0)
            pltpu.async_copy(pk_hbm.at[jnp.minimum(j0 + 3, last)], pk_c1,
                             psem1)
            pltpu.make_async_copy(g_hbm.at[row_c1], rows_v1, sem1).wait()
            pltpu.sync_copy(rows_v1, acc.at[col_c1], add=True)
            return carry

        lax.fori_loop(0, cnt // 2, body, 0)
        pltpu.make_async_copy(g_hbm.at[row_c0], rows_v0, sem0).wait()
        pltpu.make_async_copy(pk_hbm.at[last], pk_c1, psem1).wait()
        plsc.subcore_barrier()
        pltpu.sync_copy(acc.at[pl.ds(s * RPT, RPT)],
                        out_hbm.at[c, pl.ds(s * RPT, RPT)])

    return k(g_full, pk_mat)


_BLK = 2048


def _dinv_of(degp_ref):
    d = degp_ref[0][:, 0:1] + degp_ref[1][:, 0:1] + 1.0
    return lax.rsqrt(d)


def _tc_first(x_pad, importance, W1, degp):
    def body(x_ref, imp_ref, w_ref, degp_ref, out_ref):
        dinv = _dinv_of(degp_ref)
        h = x_ref[...] * imp_ref[...]
        t = lax.dot_general(h, w_ref[...], (((1,), (1,)), ((), ())),
                            preferred_element_type=jnp.float32)
        out_ref[...] = t * dinv

    return pl.pallas_call(
        body,
        grid=(N_PAD // _BLK,),
        in_specs=[
            pl.BlockSpec((_BLK, D), lambda i: (i, 0)),
            pl.BlockSpec((1, D), lambda i: (0, 0)),
            pl.BlockSpec((H, D), lambda i: (0, 0)),
            pl.BlockSpec((2, _BLK, 16), lambda i: (0, i, 0)),
        ],
        out_specs=pl.BlockSpec((_BLK, H), lambda i: (i, 0)),
        out_shape=jax.ShapeDtypeStruct((N_PAD, H), jnp.float32),
    )(x_pad, importance.reshape(1, D), W1, degp)


def _tc_mid(p, g_prev, degp, b, W):
    def body(p_ref, g_ref, degp_ref, b_ref, w_ref, out_ref):
        dinv = _dinv_of(degp_ref)
        s = (p_ref[0] + p_ref[1] + g_ref[...]) * dinv + b_ref[...]
        h = _gelu(s)
        t = lax.dot_general(h, w_ref[...], (((1,), (1,)), ((), ())),
                            preferred_element_type=jnp.float32)
        out_ref[...] = t * dinv

    return pl.pallas_call(
        body,
        grid=(N_PAD // _BLK,),
        in_specs=[
            pl.BlockSpec((2, _BLK, H), lambda i: (0, i, 0)),
            pl.BlockSpec((_BLK, H), lambda i: (i, 0)),
            pl.BlockSpec((2, _BLK, 16), lambda i: (0, i, 0)),
            pl.BlockSpec((1, H), lambda i: (0, 0)),
            pl.BlockSpec((H, H), lambda i: (0, 0)),
        ],
        out_specs=pl.BlockSpec((_BLK, H), lambda i: (i, 0)),
        out_shape=jax.ShapeDtypeStruct((N_PAD, H), jnp.float32),
    )(p, g_prev, degp, b.reshape(1, H), W)


def _tc_head(p, g_prev, degp, b3, lin1_W, lin1_b, lin2_W, lin2_b):
    def body(p_ref, g_ref, degp_ref, b3_ref, w1_ref, b1_ref, w2_ref, b2_ref,
             out_ref):
        dinv = _dinv_of(degp_ref)
        s = (p_ref[0] + p_ref[1] + g_ref[...]) * dinv + b3_ref[...]
        h = _gelu(s)
        h = _gelu(lax.dot_general(h, w1_ref[...], (((1,), (1,)), ((), ())),
                                  preferred_element_type=jnp.float32)
                  + b1_ref[...])
        out_ref[...] = lax.dot_general(
            h, w2_ref[...], (((1,), (1,)), ((), ())),
            preferred_element_type=jnp.float32) + b2_ref[...]

    return pl.pallas_call(
        body,
        grid=(N_PAD // _BLK,),
        in_specs=[
            pl.BlockSpec((2, _BLK, H), lambda i: (0, i, 0)),
            pl.BlockSpec((_BLK, H), lambda i: (i, 0)),
            pl.BlockSpec((2, _BLK, 16), lambda i: (0, i, 0)),
            pl.BlockSpec((1, H), lambda i: (0, 0)),
            pl.BlockSpec((H, H), lambda i: (0, 0)),
            pl.BlockSpec((1, H), lambda i: (0, 0)),
            pl.BlockSpec((C, H), lambda i: (0, 0)),
            pl.BlockSpec((1, C), lambda i: (0, 0)),
        ],
        out_specs=pl.BlockSpec((_BLK, C), lambda i: (i, 0)),
        out_shape=jax.ShapeDtypeStruct((N_PAD, C), jnp.float32),
    )(p, g_prev, degp, b3.reshape(1, H), lin1_W, lin1_b.reshape(1, H),
      lin2_W, lin2_b.reshape(1, C))


def kernel(x, edge_index, importance, W1, b1, W2, b2, W3, b3,
           lin1_W, lin1_b, lin2_W, lin2_b):
    x_pad = jnp.zeros((N_PAD, D), jnp.float32).at[:N].set(x)
    pad_e = E_PAD - E
    packed = jnp.left_shift(edge_index[0], 14) | edge_index[1]
    dummy = jnp.full((pad_e,), (N << 14) | N, jnp.int32)
    pk_mat = jnp.concatenate([packed, dummy]).reshape(E_PAD // CH, CH)

    degp = _sc_degree(pk_mat)[:, :, :16]
    g = _tc_first(x_pad, importance, W1, degp)
    p = _sc_scatter(g, pk_mat)
    g2 = _tc_mid(p, g, degp, b1, W2)
    p = _sc_scatter(g2, pk_mat)
    g3 = _tc_mid(p, g2, degp, b2, W3)
    p = _sc_scatter(g3, pk_mat)
    out = _tc_head(p, g3, degp, b3, lin1_W, lin1_b, lin2_W, lin2_b)
    return out[:N]


# speedup vs baseline: 1.2870x; 1.2870x over previous
import functools

import jax
import jax.numpy as jnp
from jax import lax
from jax.experimental import pallas as pl
from jax.experimental.pallas import tpu as pltpu
from jax.experimental.pallas import tpu_sc as plsc

N = 10000
D = 128
H = 128
C = 64
N_PAD = 10240
E = 320000
CH = 128
N_TILES = 32
E_PAD = 327680
CPT = E_PAD // (N_TILES * CH)
CPG = 2 * CPT
A_C0 = 130
RPT = N_PAD // 16

_SQRT_HALF = 0.7071067811865476


def _gelu(x):
    return x * 0.5 * (1.0 + lax.erf(x * _SQRT_HALF))


def _mesh():
    return plsc.VectorSubcoreMesh(core_axis_name="c", subcore_axis_name="s")


def _unpack_edges(pk_v, row_v, col_v, want_row):
    def body(j, carry):
        for kk in range(CH // 16):
            v = pk_v[j, pl.ds(kk * 16, 16)]
            if want_row:
                row_v[j, pl.ds(kk * 16, 16)] = lax.shift_right_logical(v, 14)
            col_v[j, pl.ds(kk * 16, 16)] = lax.bitwise_and(v, 16383)
        return carry

    lax.fori_loop(0, CPT, body, 0)


def _sc_degree(pk_mat):
    @functools.partial(
        pl.kernel,
        out_type=jax.ShapeDtypeStruct((2, N_PAD, H), jnp.float32),
        mesh=_mesh(),
        scratch_types=[
            pltpu.VMEM((CPT, CH), jnp.int32),
            pltpu.VMEM((CPT, CH), jnp.int32),
            pltpu.VMEM((CH, H), jnp.float32),
            pltpu.VMEM_SHARED((N_PAD, H), jnp.float32),
        ],
    )
    def k(pk_hbm, out_hbm, pk_v, col_v, ones_v, acc):
        c = lax.axis_index("c")
        s = lax.axis_index("s")
        wid = c * 16 + s

        def set_zero(i, carry):
            for kk in range(H // 16):
                ones_v[i, pl.ds(kk * 16, 16)] = jnp.zeros((16,), jnp.float32)
            return carry

        lax.fori_loop(0, CH, set_zero, 0)
        for kk in range(RPT // CH):
            pltpu.sync_copy(ones_v, acc.at[pl.ds(s * RPT + kk * CH, CH)])

        def set_ones(i, carry):
            ones_v[i, pl.ds(0, 16)] = jnp.ones((16,), jnp.float32)
            return carry

        lax.fori_loop(0, CH, set_ones, 0)
        pltpu.sync_copy(pk_hbm.at[pl.ds(wid * CPT, CPT)], pk_v)
        _unpack_edges(pk_v, None, col_v, want_row=False)
        plsc.subcore_barrier()

        def body(j, carry):
            pltpu.sync_copy(ones_v, acc.at[col_v.at[j]], add=True)
            return carry

        lax.fori_loop(0, CPT, body, 0)
        plsc.subcore_barrier()
        pltpu.sync_copy(acc.at[pl.ds(s * RPT, RPT)],
                        out_hbm.at[c, pl.ds(s * RPT, RPT)])

    return k(pk_mat)


def _sc_scatter(g_full, pk_mat):
    @functools.partial(
        pl.kernel,
        out_type=jax.ShapeDtypeStruct((2, N_PAD, H), jnp.float32),
        mesh=_mesh(),
        scratch_types=[
            pltpu.VMEM((CH,), jnp.int32),
            pltpu.VMEM((CH,), jnp.int32),
            pltpu.VMEM((CH,), jnp.int32),
            pltpu.VMEM((CH,), jnp.int32),
            pltpu.VMEM((CH,), jnp.int32),
            pltpu.VMEM((CH,), jnp.int32),
            pltpu.VMEM((CH, H), jnp.float32),
            pltpu.VMEM((CH, H), jnp.float32),
            pltpu.VMEM_SHARED((N_PAD, H), jnp.float32),
            pltpu.SemaphoreType.DMA,
            pltpu.SemaphoreType.DMA,
            pltpu.SemaphoreType.DMA,
            pltpu.SemaphoreType.DMA,
        ],
    )
    def k(g_hbm, pk_hbm, out_hbm,
          pk_c0, pk_c1, row_c0, col_c0, row_c1, col_c1,
          rows_v0, rows_v1, acc, sem0, sem1, psem0, psem1):
        c = lax.axis_index("c")
        s = lax.axis_index("s")
        base = s * CPG + c * A_C0
        cnt = jnp.where(c == 0, A_C0, CPG - A_C0)
        last = base + cnt - 1

        def unpack(pk_c, row_c, col_c):
            for kk in range(CH // 16):
                v = pk_c[pl.ds(kk * 16, 16)]
                row_c[pl.ds(kk * 16, 16)] = lax.shift_right_logical(v, 14)
                col_c[pl.ds(kk * 16, 16)] = lax.bitwise_and(v, 16383)

        def zero_rows(i, carry):
            for kk in range(H // 16):
                rows_v0[i, pl.ds(kk * 16, 16)] = jnp.zeros((16,),
                                                           jnp.float32)
            return carry

        lax.fori_loop(0, CH, zero_rows, 0)
        for kk in range(RPT // CH):
            pltpu.sync_copy(rows_v0, acc.at[pl.ds(s * RPT + kk * CH, CH)])
        plsc.subcore_barrier()

        pltpu.sync_copy(pk_hbm.at[base], pk_c0)
        unpack(pk_c0, row_c0, col_c0)
        pltpu.async_copy(g_hbm.at[row_c0], rows_v0, sem0)
        pltpu.async_copy(pk_hbm.at[base + 1], pk_c1, psem1)

        def body(i, carry):
            j0 = base + i * 2
            pltpu.make_async_copy(pk_hbm.at[j0 + 1], pk_c1, psem1).wait()
            unpack(pk_c1, row_c1, col_c1)
            pltpu.async_copy(g_hbm.at[row_c1], rows_v1, sem1)
            pltpu.async_copy(pk_hbm.at[jnp.minimum(j0 + 2, last)], pk_c0,
                             psem0)
            pltpu.make_async_copy(g_hbm.at[row_c0], rows_v0, sem0).wait()
            pltpu.sync_copy(rows_v0, acc.at[col_c0], add=True)
            pltpu.make_async_copy(pk_hbm.at[j0], pk_c0, psem0).wait()
            unpack(pk_c0, row_c0, col_c0)
            pltpu.async_copy(g_hbm.at[row_c0], rows_v0, sem0)
            pltpu.async_copy(pk_hbm.at[jnp.minimum(j0 + 3, last)], pk_c1,
                             psem1)
            pltpu.make_async_copy(g_hbm.at[row_c1], rows_v1, sem1).wait()
            pltpu.sync_copy(rows_v1, acc.at[col_c1], add=True)
            return carry

        lax.fori_loop(0, cnt // 2, body, 0)
        pltpu.make_async_copy(g_hbm.at[row_c0], rows_v0, sem0).wait()
        pltpu.make_async_copy(pk_hbm.at[last], pk_c1, psem1).wait()
        plsc.subcore_barrier()
        pltpu.sync_copy(acc.at[pl.ds(s * RPT, RPT)],
                        out_hbm.at[c, pl.ds(s * RPT, RPT)])

    return k(g_full, pk_mat)


_BLK = 2048


def _dinv_of(degp_ref):
    d = degp_ref[0][:, 0:1] + degp_ref[1][:, 0:1] + 1.0
    return lax.rsqrt(d)


def _tc_first(x_pad, importance, W1, degp):
    def body(x_ref, imp_ref, w_ref, degp_ref, out_ref):
        dinv = _dinv_of(degp_ref)
        h = x_ref[...] * imp_ref[...]
        t = lax.dot_general(h, w_ref[...], (((1,), (1,)), ((), ())),
                            preferred_element_type=jnp.float32)
        out_ref[...] = t * dinv

    return pl.pallas_call(
        body,
        grid=(N_PAD // _BLK,),
        in_specs=[
            pl.BlockSpec((_BLK, D), lambda i: (i, 0)),
            pl.BlockSpec((1, D), lambda i: (0, 0)),
            pl.BlockSpec((H, D), lambda i: (0, 0)),
            pl.BlockSpec((2, _BLK, 16), lambda i: (0, i, 0)),
        ],
        out_specs=pl.BlockSpec((_BLK, H), lambda i: (i, 0)),
        out_shape=jax.ShapeDtypeStruct((N_PAD, H), jnp.float32),
    )(x_pad, importance.reshape(1, D), W1, degp)


def _tc_mid(p, g_prev, degp, b, W):
    def body(p_ref, g_ref, degp_ref, b_ref, w_ref, out_ref):
        dinv = _dinv_of(degp_ref)
        s = (p_ref[0] + p_ref[1] + g_ref[...]) * dinv + b_ref[...]
        h = _gelu(s)
        t = lax.dot_general(h, w_ref[...], (((1,), (1,)), ((), ())),
                            preferred_element_type=jnp.float32)
        out_ref[...] = t * dinv

    return pl.pallas_call(
        body,
        grid=(N_PAD // _BLK,),
        in_specs=[
            pl.BlockSpec((2, _BLK, H), lambda i: (0, i, 0)),
            pl.BlockSpec((_BLK, H), lambda i: (i, 0)),
            pl.BlockSpec((2, _BLK, 16), lambda i: (0, i, 0)),
            pl.BlockSpec((1, H), lambda i: (0, 0)),
            pl.BlockSpec((H, H), lambda i: (0, 0)),
        ],
        out_specs=pl.BlockSpec((_BLK, H), lambda i: (i, 0)),
        out_shape=jax.ShapeDtypeStruct((N_PAD, H), jnp.float32),
    )(p, g_prev, degp, b.reshape(1, H), W)


def _tc_head(p, g_prev, degp, b3, lin1_W, lin1_b, lin2_W, lin2_b):
    def body(p_ref, g_ref, degp_ref, b3_ref, w1_ref, b1_ref, w2_ref, b2_ref,
             out_ref):
        dinv = _dinv_of(degp_ref)
        s = (p_ref[0] + p_ref[1] + g_ref[...]) * dinv + b3_ref[...]
        h = _gelu(s)
        h = _gelu(lax.dot_general(h, w1_ref[...], (((1,), (1,)), ((), ())),
                                  preferred_element_type=jnp.float32)
                  + b1_ref[...])
        out_ref[...] = lax.dot_general(
            h, w2_ref[...], (((1,), (1,)), ((), ())),
            preferred_element_type=jnp.float32) + b2_ref[...]

    return pl.pallas_call(
        body,
        grid=(N_PAD // _BLK,),
        in_specs=[
            pl.BlockSpec((2, _BLK, H), lambda i: (0, i, 0)),
            pl.BlockSpec((_BLK, H), lambda i: (i, 0)),
            pl.BlockSpec((2, _BLK, 16), lambda i: (0, i, 0)),
            pl.BlockSpec((1, H), lambda i: (0, 0)),
            pl.BlockSpec((H, H), lambda i: (0, 0)),
            pl.BlockSpec((1, H), lambda i: (0, 0)),
            pl.BlockSpec((C, H), lambda i: (0, 0)),
            pl.BlockSpec((1, C), lambda i: (0, 0)),
        ],
        out_specs=pl.BlockSpec((_BLK, C), lambda i: (i, 0)),
        out_shape=jax.ShapeDtypeStruct((N_PAD, C), jnp.float32),
    )(p, g_prev, degp, b3.reshape(1, H), lin1_W, lin1_b.reshape(1, H),
      lin2_W, lin2_b.reshape(1, C))


def kernel(x, edge_index, importance, W1, b1, W2, b2, W3, b3,
           lin1_W, lin1_b, lin2_W, lin2_b):
    x_pad = jnp.zeros((N_PAD, D), jnp.float32).at[:N].set(x)
    pad_e = E_PAD - E
    packed = jnp.left_shift(edge_index[0], 14) | edge_index[1]
    dummy = jnp.full((pad_e,), (N << 14) | N, jnp.int32)
    pk_mat = jnp.concatenate([packed, dummy]).reshape(E_PAD // CH, CH)

    degp = _sc_degree(pk_mat)[:, :, :16]
    g = _tc_first(x_pad, importance, W1, degp)
    p = _sc_scatter(g, pk_mat)
    g2 = _tc_mid(p, g, degp, b1, W2)
    p = _sc_scatter(g2, pk_mat)
    g3 = _tc_mid(p, g2, degp, b2, W3)
    p = _sc_scatter(g3, pk_mat)
    out = _tc_head(p, g3, degp, b3, lin1_W, lin1_b, lin2_W, lin2_b)
    return out[:N]


# speedup vs baseline: 1.3415x; 1.0424x over previous
import functools

import jax
import jax.numpy as jnp
from jax import lax
from jax.experimental import pallas as pl
from jax.experimental.pallas import tpu as pltpu
from jax.experimental.pallas import tpu_sc as plsc

N = 10000
D = 128
H = 128
C = 64
N_PAD = 10112
E = 320000
CH = 128
N_TILES = 32
E_PAD = 327680
CPT = E_PAD // (N_TILES * CH)
CPG = 2 * CPT
A_C0 = 128
MAXC = max(A_C0, CPG - A_C0)
RPT = N_PAD // 16

_SQRT_HALF = 0.7071067811865476


def _gelu(x):
    return x * 0.5 * (1.0 + lax.erf(x * _SQRT_HALF))


def _mesh():
    return plsc.VectorSubcoreMesh(core_axis_name="c", subcore_axis_name="s")


def _unpack_edges(pk_v, row_v, col_v, want_row):
    def body(j, carry):
        for kk in range(CH // 16):
            v = pk_v[j, pl.ds(kk * 16, 16)]
            if want_row:
                row_v[j, pl.ds(kk * 16, 16)] = lax.shift_right_logical(v, 14)
            col_v[j, pl.ds(kk * 16, 16)] = lax.bitwise_and(v, 16383)
        return carry

    lax.fori_loop(0, CPT, body, 0)


def _sc_degree(pk_mat):
    @functools.partial(
        pl.kernel,
        out_type=jax.ShapeDtypeStruct((2, N_PAD, H), jnp.float32),
        mesh=_mesh(),
        scratch_types=[
            pltpu.VMEM((CPT, CH), jnp.int32),
            pltpu.VMEM((CPT, CH), jnp.int32),
            pltpu.VMEM((CH, H), jnp.float32),
            pltpu.VMEM_SHARED((N_PAD, H), jnp.float32),
        ],
    )
    def k(pk_hbm, out_hbm, pk_v, col_v, ones_v, acc):
        c = lax.axis_index("c")
        s = lax.axis_index("s")
        wid = c * 16 + s

        def set_zero(i, carry):
            for kk in range(H // 16):
                ones_v[i, pl.ds(kk * 16, 16)] = jnp.zeros((16,), jnp.float32)
            return carry

        lax.fori_loop(0, CH, set_zero, 0)
        for kk in range(RPT // CH):
            pltpu.sync_copy(ones_v, acc.at[pl.ds(s * RPT + kk * CH, CH)])
        if RPT % CH:
            pltpu.sync_copy(
                ones_v.at[pl.ds(0, RPT % CH)],
                acc.at[pl.ds(s * RPT + (RPT // CH) * CH, RPT % CH)])

        def set_ones(i, carry):
            ones_v[i, pl.ds(0, 16)] = jnp.ones((16,), jnp.float32)
            return carry

        lax.fori_loop(0, CH, set_ones, 0)
        pltpu.sync_copy(pk_hbm.at[pl.ds(wid * CPT, CPT)], pk_v)
        _unpack_edges(pk_v, None, col_v, want_row=False)
        plsc.subcore_barrier()

        def body(j, carry):
            pltpu.sync_copy(ones_v, acc.at[col_v.at[j]], add=True)
            return carry

        lax.fori_loop(0, CPT, body, 0)
        plsc.subcore_barrier()
        pltpu.sync_copy(acc.at[pl.ds(s * RPT, RPT)],
                        out_hbm.at[c, pl.ds(s * RPT, RPT)])

    return k(pk_mat)


def _sc_scatter(g_full, pk_mat):
    @functools.partial(
        pl.kernel,
        out_type=jax.ShapeDtypeStruct((2, N_PAD, H), jnp.float32),
        mesh=_mesh(),
        scratch_types=[
            pltpu.VMEM((MAXC, CH), jnp.int32),
            pltpu.VMEM((CH,), jnp.int32),
            pltpu.VMEM((CH,), jnp.int32),
            pltpu.VMEM((CH,), jnp.int32),
            pltpu.VMEM((CH,), jnp.int32),
            pltpu.VMEM((CH, H), jnp.float32),
            pltpu.VMEM((CH, H), jnp.float32),
            pltpu.VMEM_SHARED((N_PAD, H), jnp.float32),
            pltpu.SemaphoreType.DMA,
            pltpu.SemaphoreType.DMA,
        ],
    )
    def k(g_hbm, pk_hbm, out_hbm,
          pk_v, row_c0, col_c0, row_c1, col_c1,
          rows_v0, rows_v1, acc, sem0, sem1):
        c = lax.axis_index("c")
        s = lax.axis_index("s")
        base = s * CPG + c * A_C0
        cnt = jnp.where(c == 0, A_C0, CPG - A_C0)

        def unpack(j, row_c, col_c):
            for kk in range(CH // 16):
                v = pk_v[j, pl.ds(kk * 16, 16)]
                row_c[pl.ds(kk * 16, 16)] = lax.shift_right_logical(v, 14)
                col_c[pl.ds(kk * 16, 16)] = lax.bitwise_and(v, 16383)

        def zero_rows(i, carry):
            for kk in range(H // 16):
                rows_v0[i, pl.ds(kk * 16, 16)] = jnp.zeros((16,),
                                                           jnp.float32)
            return carry

        lax.fori_loop(0, CH, zero_rows, 0)
        for kk in range(RPT // CH):
            pltpu.sync_copy(rows_v0, acc.at[pl.ds(s * RPT + kk * CH, CH)])
        if RPT % CH:
            pltpu.sync_copy(
                rows_v0.at[pl.ds(0, RPT % CH)],
                acc.at[pl.ds(s * RPT + (RPT // CH) * CH, RPT % CH)])
        pltpu.sync_copy(pk_hbm.at[pl.ds(base, MAXC)], pk_v)
        plsc.subcore_barrier()

        unpack(0, row_c0, col_c0)
        pltpu.async_copy(g_hbm.at[row_c0], rows_v0, sem0)

        def body(i, carry):
            j0 = i * 2
            unpack(j0 + 1, row_c1, col_c1)
            pltpu.async_copy(g_hbm.at[row_c1], rows_v1, sem1)
            pltpu.make_async_copy(g_hbm.at[row_c0], rows_v0, sem0).wait()
            pltpu.sync_copy(rows_v0, acc.at[col_c0], add=True)
            unpack(jnp.minimum(j0 + 2, cnt - 1), row_c0, col_c0)
            pltpu.async_copy(g_hbm.at[row_c0], rows_v0, sem0)
            pltpu.make_async_copy(g_hbm.at[row_c1], rows_v1, sem1).wait()
            pltpu.sync_copy(rows_v1, acc.at[col_c1], add=True)
            return carry

        lax.fori_loop(0, cnt // 2, body, 0)
        pltpu.make_async_copy(g_hbm.at[row_c0], rows_v0, sem0).wait()
        plsc.subcore_barrier()
        pltpu.sync_copy(acc.at[pl.ds(s * RPT, RPT)],
                        out_hbm.at[c, pl.ds(s * RPT, RPT)])

    return k(g_full, pk_mat)


_BLK = 1264


def _dinv_of(degp_ref):
    d = degp_ref[0][:, 0:1] + degp_ref[1][:, 0:1] + 1.0
    return lax.rsqrt(d)


def _tc_first(x_pad, importance, W1, degp):
    def body(x_ref, imp_ref, w_ref, degp_ref, out_ref):
        dinv = _dinv_of(degp_ref)
        h = x_ref[...] * imp_ref[...]
        t = lax.dot_general(h, w_ref[...], (((1,), (1,)), ((), ())),
                            preferred_element_type=jnp.float32)
        out_ref[...] = t * dinv

    return pl.pallas_call(
        body,
        grid=(N_PAD // _BLK,),
        in_specs=[
            pl.BlockSpec((_BLK, D), lambda i: (i, 0)),
            pl.BlockSpec((1, D), lambda i: (0, 0)),
            pl.BlockSpec((H, D), lambda i: (0, 0)),
            pl.BlockSpec((2, _BLK, 16), lambda i: (0, i, 0)),
        ],
        out_specs=pl.BlockSpec((_BLK, H), lambda i: (i, 0)),
        out_shape=jax.ShapeDtypeStruct((N_PAD, H), jnp.float32),
    )(x_pad, importance.reshape(1, D), W1, degp)


def _tc_mid(p, g_prev, degp, b, W):
    def body(p_ref, g_ref, degp_ref, b_ref, w_ref, out_ref):
        dinv = _dinv_of(degp_ref)
        s = (p_ref[0] + p_ref[1] + g_ref[...]) * dinv + b_ref[...]
        h = _gelu(s)
        t = lax.dot_general(h, w_ref[...], (((1,), (1,)), ((), ())),
                            preferred_element_type=jnp.float32)
        out_ref[...] = t * dinv

    return pl.pallas_call(
        body,
        grid=(N_PAD // _BLK,),
        in_specs=[
            pl.BlockSpec((2, _BLK, H), lambda i: (0, i, 0)),
            pl.BlockSpec((_BLK, H), lambda i: (i, 0)),
            pl.BlockSpec((2, _BLK, 16), lambda i: (0, i, 0)),
            pl.BlockSpec((1, H), lambda i: (0, 0)),
            pl.BlockSpec((H, H), lambda i: (0, 0)),
        ],
        out_specs=pl.BlockSpec((_BLK, H), lambda i: (i, 0)),
        out_shape=jax.ShapeDtypeStruct((N_PAD, H), jnp.float32),
    )(p, g_prev, degp, b.reshape(1, H), W)


def _tc_head(p, g_prev, degp, b3, lin1_W, lin1_b, lin2_W, lin2_b):
    def body(p_ref, g_ref, degp_ref, b3_ref, w1_ref, b1_ref, w2_ref, b2_ref,
             out_ref):
        dinv = _dinv_of(degp_ref)
        s = (p_ref[0] + p_ref[1] + g_ref[...]) * dinv + b3_ref[...]
        h = _gelu(s)
        h = _gelu(lax.dot_general(h, w1_ref[...], (((1,), (1,)), ((), ())),
                                  preferred_element_type=jnp.float32)
                  + b1_ref[...])
        out_ref[...] = lax.dot_general(
            h, w2_ref[...], (((1,), (1,)), ((), ())),
            preferred_element_type=jnp.float32) + b2_ref[...]

    return pl.pallas_call(
        body,
        grid=(N_PAD // _BLK,),
        in_specs=[
            pl.BlockSpec((2, _BLK, H), lambda i: (0, i, 0)),
            pl.BlockSpec((_BLK, H), lambda i: (i, 0)),
            pl.BlockSpec((2, _BLK, 16), lambda i: (0, i, 0)),
            pl.BlockSpec((1, H), lambda i: (0, 0)),
            pl.BlockSpec((H, H), lambda i: (0, 0)),
            pl.BlockSpec((1, H), lambda i: (0, 0)),
            pl.BlockSpec((C, H), lambda i: (0, 0)),
            pl.BlockSpec((1, C), lambda i: (0, 0)),
        ],
        out_specs=pl.BlockSpec((_BLK, C), lambda i: (i, 0)),
        out_shape=jax.ShapeDtypeStruct((N_PAD, C), jnp.float32),
    )(p, g_prev, degp, b3.reshape(1, H), lin1_W, lin1_b.reshape(1, H),
      lin2_W, lin2_b.reshape(1, C))


def kernel(x, edge_index, importance, W1, b1, W2, b2, W3, b3,
           lin1_W, lin1_b, lin2_W, lin2_b):
    x_pad = jnp.zeros((N_PAD, D), jnp.float32).at[:N].set(x)
    pad_e = E_PAD - E
    packed = jnp.left_shift(edge_index[0], 14) | edge_index[1]
    pad_e += MAXC * CH
    dummy = jnp.full((pad_e,), (N << 14) | N, jnp.int32)
    pk_mat = jnp.concatenate([packed, dummy]).reshape(
        E_PAD // CH + MAXC, CH)

    degp = _sc_degree(pk_mat)[:, :, :16]
    g = _tc_first(x_pad, importance, W1, degp)
    p = _sc_scatter(g, pk_mat)
    g2 = _tc_mid(p, g, degp, b1, W2)
    p = _sc_scatter(g2, pk_mat)
    g3 = _tc_mid(p, g2, degp, b2, W3)
    p = _sc_scatter(g3, pk_mat)
    out = _tc_head(p, g3, degp, b3, lin1_W, lin1_b, lin2_W, lin2_b)
    return out[:N]
